# baseline (speedup 1.0000x reference)
import functools

import jax
import jax.numpy as jnp
from jax import lax
from jax.experimental import pallas as pl
from jax.experimental.pallas import tpu as pltpu
from jax.experimental.pallas import tpu_sc as plsc

D_MODEL = 64
SCALE = 8.0
VOCAB = 1000000

NC, NS = 2, 16
NW = NC * NS
ROWS, COLS = 4096, 200
CHUNK = 128
SG = 2
NB = 2
NITER = COLS // SG
PITCH = 137

_mesh = plsc.VectorSubcoreMesh(core_axis_name="c", subcore_axis_name="s")


@functools.partial(
    pl.kernel,
    out_type=jax.ShapeDtypeStruct((COLS, 8, NW, 8, CHUNK), jnp.float32),
    mesh=_mesh,
    scratch_types=[
        pltpu.VMEM((COLS, CHUNK), jnp.int32),
        pltpu.VMEM((NB, SG * CHUNK), jnp.int32),
        pltpu.VMEM((NB, SG * CHUNK, CHUNK), jnp.float32),
        pltpu.VMEM((NB, SG, 8, 8, PITCH), jnp.float32),
    ] + [pltpu.SemaphoreType.DMA] * (2 * NB),
    compiler_params=pltpu.CompilerParams(
        use_tc_tiling_on_sc=False, needs_layout_passes=False),
)
def _emb_lookup(x_hbm, lutp_hbm, out_hbm, idx_v, idx2_v, gbuf, obuf, *sems):
    gsem, wsem = sems[:NB], sems[NB:]
    wid = lax.axis_index("s") * NC + lax.axis_index("c")
    pltpu.sync_copy(x_hbm.at[wid], idx_v)

    lanes = lax.iota(jnp.int32, 16)
    drv = jnp.bitwise_and(lanes, 7)
    dtvs = [jnp.right_shift(c * 16 + lanes, 3) for c in range(4)]
    cbase = [c * 16 + lanes for c in range(4)]

    def start_gathers(J, b):
        for t in range(SG):
            for u in range(CHUNK // 16):
                s = pl.ds(u * 16, 16)
                st = pl.ds(t * CHUNK + u * 16, 16)
                idx2_v[b, st] = jnp.right_shift(idx_v[J * SG + t, s], 1)
            pltpu.async_copy(
                lutp_hbm.at[idx2_v.at[b, pl.ds(t * CHUNK, CHUNK)]],
                gbuf.at[b, pl.ds(t * CHUNK, CHUNK)], gsem[b])

    def wait_gathers(b):
        pltpu.make_async_copy(
            lutp_hbm.at[pl.ds(0, SG * CHUNK)], gbuf.at[b], gsem[b]).wait()

    for b in range(NB):
        start_gathers(b, b)

    @pl.loop(0, NITER, step=NB)
    def _grp(J0):
        for b in range(NB):
            J = J0 + b
            wait_gathers(b)

            @pl.when(J >= NB)
            def _wait_write():
                pltpu.make_async_copy(
                    obuf.at[b, :, :, :, pl.ds(0, CHUNK)],
                    out_hbm.at[pl.ds((J - NB) * SG, SG), :, wid],
                    wsem[b]).wait()

            for t in range(SG):
                @plsc.parallel_loop(0, CHUNK // 16, unroll=2)
                def _g(g):
                    idxv = idx_v[J * SG + t, pl.ds(g * 16, 16)]
                    par = jnp.left_shift(jnp.bitwise_and(idxv, 1), 6)
                    for kk in range(16):
                        sel = jnp.full((16, 1), kk, jnp.int32)
                        parb = lax.gather(
                            par, sel,
                            lax.GatherDimensionNumbers(
                                offset_dims=(), collapsed_slice_dims=(0,),
                                start_index_map=(0,)),
                            (1,), mode=lax.GatherScatterMode.PROMISE_IN_BOUNDS)
                        kcol = jnp.full((16,), 0, jnp.int32) + (g * 16 + kk)
                        krow = kcol + t * CHUNK
                        for c in range(D_MODEL // 16):
                            v = plsc.load_gather(
                                gbuf.at[b], [krow, parb + cbase[c]])
                            plsc.store_scatter(
                                obuf.at[b, t], [dtvs[c], drv, kcol], v * SCALE)

            pltpu.async_copy(
                obuf.at[b, :, :, :, pl.ds(0, CHUNK)],
                out_hbm.at[pl.ds(J * SG, SG), :, wid], wsem[b])

            @pl.when(J + NB < NITER)
            def _next_gather():
                start_gathers(J + NB, b)

    for b in range(NB):
        pltpu.make_async_copy(
            obuf.at[b, :, :, :, pl.ds(0, CHUNK)],
            out_hbm.at[pl.ds((NITER - NB + b) * SG, SG), :, wid],
            wsem[b]).wait()


def kernel(x, lut):
    xw = jnp.transpose(x.reshape(NW, CHUNK, COLS), (0, 2, 1))
    lutp = lut.reshape(VOCAB // 2, 2 * D_MODEL)
    out = _emb_lookup(xw, lutp)
    final = jnp.transpose(out, (2, 4, 0, 1, 3)).reshape(ROWS, COLS, D_MODEL)
    return final


# speedup vs baseline: 1.5767x; 1.5767x over previous
import functools

import jax
import jax.numpy as jnp
from jax import lax
from jax.experimental import pallas as pl
from jax.experimental.pallas import tpu as pltpu
from jax.experimental.pallas import tpu_sc as plsc

D_MODEL = 64
SCALE = 8.0
VOCAB = 1000000

NC, NS = 2, 16
NW = NC * NS
ROWS, COLS = 4096, 200
CHUNK = 128
SG = 2
NB = 2
NITER = COLS // SG
PITCH = 137

_mesh = plsc.VectorSubcoreMesh(core_axis_name="c", subcore_axis_name="s")


@functools.partial(
    pl.kernel,
    out_type=jax.ShapeDtypeStruct((COLS, 8, NW, 8, CHUNK), jnp.float32),
    mesh=_mesh,
    scratch_types=[
        pltpu.VMEM((COLS, CHUNK), jnp.int32),
        pltpu.VMEM((NB, SG * CHUNK, D_MODEL), jnp.float32),
        pltpu.VMEM((NB, SG, 8, 8, PITCH), jnp.float32),
    ] + [pltpu.SemaphoreType.DMA] * (2 * NB),
    compiler_params=pltpu.CompilerParams(
        use_tc_tiling_on_sc=False, needs_layout_passes=False),
)
def _emb_lookup(x_hbm, lutp_hbm, out_hbm, idx_v, gbuf, obuf, *sems):
    gsem, wsem = sems[:NB], sems[NB:]
    wid = lax.axis_index("s") * NC + lax.axis_index("c")
    pltpu.sync_copy(x_hbm.at[wid], idx_v)

    lanes = lax.iota(jnp.int32, 16)
    drv = jnp.bitwise_and(lanes, 7)
    dtvs = [jnp.right_shift(c * 16 + lanes, 3) for c in range(4)]

    def start_gathers(J, b):
        for t in range(SG):
            pltpu.async_copy(
                lutp_hbm.at[idx_v.at[J * SG + t]],
                gbuf.at[b, pl.ds(t * CHUNK, CHUNK)], gsem[b])

    def wait_gathers(b):
        pltpu.make_async_copy(
            lutp_hbm.at[pl.ds(0, SG * CHUNK)], gbuf.at[b], gsem[b]).wait()

    for b in range(NB):
        start_gathers(b, b)

    @pl.loop(0, NITER, step=NB)
    def _grp(J0):
        for b in range(NB):
            J = J0 + b
            wait_gathers(b)

            @pl.when(J >= NB)
            def _wait_write():
                pltpu.make_async_copy(
                    obuf.at[b, :, :, :, pl.ds(0, CHUNK)],
                    out_hbm.at[pl.ds((J - NB) * SG, SG), :, wid],
                    wsem[b]).wait()

            for t in range(SG):
                @plsc.parallel_loop(0, CHUNK, unroll=8)
                def _k(k):
                    kv = jnp.full((16,), 0, jnp.int32) + k
                    for c in range(D_MODEL // 16):
                        v = gbuf[b, t * CHUNK + k, pl.ds(c * 16, 16)] * SCALE
                        plsc.store_scatter(
                            obuf.at[b, t], [dtvs[c], drv, kv], v)

            pltpu.async_copy(
                obuf.at[b, :, :, :, pl.ds(0, CHUNK)],
                out_hbm.at[pl.ds(J * SG, SG), :, wid], wsem[b])

            @pl.when(J + NB < NITER)
            def _next_gather():
                start_gathers(J + NB, b)

    for b in range(NB):
        pltpu.make_async_copy(
            obuf.at[b, :, :, :, pl.ds(0, CHUNK)],
            out_hbm.at[pl.ds((NITER - NB + b) * SG, SG), :, wid],
            wsem[b]).wait()


def kernel(x, lut):
    xw = jnp.transpose(x.reshape(NW, CHUNK, COLS), (0, 2, 1))
    out = _emb_lookup(xw, lut)
    final = jnp.transpose(out, (2, 4, 0, 1, 3)).reshape(ROWS, COLS, D_MODEL)
    return final
